# baseline (speedup 1.0000x reference)
import functools

import jax
import jax.numpy as jnp
from jax.experimental import pallas as pl

_DIM = 4096
_E = 64
_K = 8
_TBLK = 512


def _router_kernel(x_ref, w_ref, wts_ref, idx_ref, probs_ref,
                   usage_ref, z_ref, lb_ref, *, n_tok):
    logits = jax.lax.dot_general(
        x_ref[:], w_ref[:], (((1,), (1,)), ((), ())),
        preferred_element_type=jnp.float32)

    m = jnp.max(logits, axis=1, keepdims=True)
    ex = jnp.exp(logits - m)
    s = jnp.sum(ex, axis=1, keepdims=True)
    probs = ex / s
    probs_ref[:] = probs
    lse = m[:, 0] + jnp.log(s[:, 0])
    z_part = jnp.sum(lse * lse)
    u_part = jnp.sum(probs, axis=0)

    iota = jax.lax.broadcasted_iota(jnp.int32, logits.shape, 1)
    work = logits
    vals, inds = [], []
    for _ in range(_K):
        mk = jnp.max(work, axis=1, keepdims=True)
        ik = jnp.min(jnp.where(work == mk, iota, _E), axis=1, keepdims=True)
        vals.append(mk)
        inds.append(ik)
        work = jnp.where(iota == ik, -jnp.inf, work)
    v = jnp.concatenate(vals, axis=1)
    ix = jnp.concatenate(inds, axis=1)
    ev = jnp.exp(v - v[:, 0:1])
    wts_ref[:] = ev / jnp.sum(ev, axis=1, keepdims=True)
    idx_ref[:] = ix

    b = pl.program_id(0)

    z_blk = jnp.reshape(z_part, (1, 1))

    @pl.when(b == 0)
    def _init():
        usage_ref[:] = u_part[None, :]
        z_ref[:] = z_blk

    @pl.when(b > 0)
    def _acc():
        usage_ref[:] += u_part[None, :]
        z_ref[:] += z_blk

    @pl.when(b == pl.num_programs(0) - 1)
    def _fin():
        usage = usage_ref[:] / n_tok
        usage_ref[:] = usage
        lb_ref[:] = jnp.sum(usage * usage).reshape(1, 1) * float(_E)
        z_ref[:] = z_ref[:] / n_tok


def kernel(x, W):
    b, seq, dim = x.shape
    n_tok = b * seq
    xr = x.reshape(n_tok, dim)
    grid = (n_tok // _TBLK,)

    out = pl.pallas_call(
        functools.partial(_router_kernel, n_tok=float(n_tok)),
        grid=grid,
        in_specs=[
            pl.BlockSpec((_TBLK, dim), lambda i: (i, 0)),
            pl.BlockSpec((_E, dim), lambda i: (0, 0)),
        ],
        out_specs=[
            pl.BlockSpec((_TBLK, _K), lambda i: (i, 0)),
            pl.BlockSpec((_TBLK, _K), lambda i: (i, 0)),
            pl.BlockSpec((_TBLK, _E), lambda i: (i, 0)),
            pl.BlockSpec((1, _E), lambda i: (0, 0)),
            pl.BlockSpec((1, 1), lambda i: (0, 0)),
            pl.BlockSpec((1, 1), lambda i: (0, 0)),
        ],
        out_shape=[
            jax.ShapeDtypeStruct((n_tok, _K), jnp.float32),
            jax.ShapeDtypeStruct((n_tok, _K), jnp.int32),
            jax.ShapeDtypeStruct((n_tok, _E), jnp.float32),
            jax.ShapeDtypeStruct((1, _E), jnp.float32),
            jax.ShapeDtypeStruct((1, 1), jnp.float32),
            jax.ShapeDtypeStruct((1, 1), jnp.float32),
        ],
    )(xr, W)

    wts, idx, probs, usage, z, lb = out
    return (wts.reshape(b, seq, _K),
            idx.reshape(b, seq, _K),
            lb[0, 0],
            z[0, 0],
            usage[0],
            probs.reshape(b, seq, _E))


# speedup vs baseline: 1.2596x; 1.2596x over previous
import functools

import jax
import jax.numpy as jnp
from jax.experimental import pallas as pl

_DIM = 4096
_E = 64
_K = 8
_TBLK = 512


def _router_kernel(x_ref, w_ref, wts_ref, idx_ref, probs_ref,
                   usage_ref, z_ref, lb_ref, *, n_tok):
    logits = jax.lax.dot_general(
        x_ref[:], w_ref[:], (((1,), (1,)), ((), ())),
        preferred_element_type=jnp.float32)

    m = jnp.max(logits, axis=1, keepdims=True)
    ex = jnp.exp(logits - m)
    s = jnp.sum(ex, axis=1, keepdims=True)
    probs = ex / s
    probs_ref[:] = probs
    lse = m[:, 0] + jnp.log(s[:, 0])
    z_part = jnp.sum(lse * lse)
    u_part = jnp.sum(probs, axis=0)

    iota = jax.lax.broadcasted_iota(jnp.int32, logits.shape, 1)
    bits = jax.lax.bitcast_convert_type(logits, jnp.int32)
    code = jnp.where(logits >= 0.0, (_E - 1) - iota, iota)
    keys = jax.lax.bitcast_convert_type((bits & ~(_E - 1)) | code, jnp.float32)
    work = keys
    vals, inds = [], []
    for _ in range(_K):
        mk = jnp.max(work, axis=1, keepdims=True)
        vals.append(mk)
        inds.append(jax.lax.bitcast_convert_type(mk, jnp.int32) & (_E - 1))
        work = jnp.where(work == mk, -jnp.inf, work)
    v = jnp.concatenate(vals, axis=1)
    low = jnp.concatenate(inds, axis=1)
    ix = jnp.where(v >= 0.0, (_E - 1) - low, low)
    ev = jnp.exp(v - v[:, 0:1])
    wts_ref[:] = ev / jnp.sum(ev, axis=1, keepdims=True)
    idx_ref[:] = ix

    b = pl.program_id(0)

    z_blk = jnp.reshape(z_part, (1, 1))

    @pl.when(b == 0)
    def _init():
        usage_ref[:] = u_part[None, :]
        z_ref[:] = z_blk

    @pl.when(b > 0)
    def _acc():
        usage_ref[:] += u_part[None, :]
        z_ref[:] += z_blk

    @pl.when(b == pl.num_programs(0) - 1)
    def _fin():
        usage = usage_ref[:] / n_tok
        usage_ref[:] = usage
        lb_ref[:] = jnp.sum(usage * usage).reshape(1, 1) * float(_E)
        z_ref[:] = z_ref[:] / n_tok


def kernel(x, W):
    b, seq, dim = x.shape
    n_tok = b * seq
    xr = x.reshape(n_tok, dim)
    grid = (n_tok // _TBLK,)

    out = pl.pallas_call(
        functools.partial(_router_kernel, n_tok=float(n_tok)),
        grid=grid,
        in_specs=[
            pl.BlockSpec((_TBLK, dim), lambda i: (i, 0)),
            pl.BlockSpec((_E, dim), lambda i: (0, 0)),
        ],
        out_specs=[
            pl.BlockSpec((_TBLK, _K), lambda i: (i, 0)),
            pl.BlockSpec((_TBLK, _K), lambda i: (i, 0)),
            pl.BlockSpec((_TBLK, _E), lambda i: (i, 0)),
            pl.BlockSpec((1, _E), lambda i: (0, 0)),
            pl.BlockSpec((1, 1), lambda i: (0, 0)),
            pl.BlockSpec((1, 1), lambda i: (0, 0)),
        ],
        out_shape=[
            jax.ShapeDtypeStruct((n_tok, _K), jnp.float32),
            jax.ShapeDtypeStruct((n_tok, _K), jnp.int32),
            jax.ShapeDtypeStruct((n_tok, _E), jnp.float32),
            jax.ShapeDtypeStruct((1, _E), jnp.float32),
            jax.ShapeDtypeStruct((1, 1), jnp.float32),
            jax.ShapeDtypeStruct((1, 1), jnp.float32),
        ],
    )(xr, W)

    wts, idx, probs, usage, z, lb = out
    return (wts.reshape(b, seq, _K),
            idx.reshape(b, seq, _K),
            lb[0, 0],
            z[0, 0],
            usage[0],
            probs.reshape(b, seq, _E))


# speedup vs baseline: 1.3501x; 1.0719x over previous
import functools

import jax
import jax.numpy as jnp
from jax.experimental import pallas as pl

_DIM = 4096
_E = 64
_K = 8
_TBLK = 1024


def _router_kernel(x_ref, w_ref, wts_ref, idx_ref, probs_ref,
                   usage_ref, z_ref, lb_ref, *, n_tok):
    logits = jax.lax.dot_general(
        x_ref[:], w_ref[:], (((1,), (1,)), ((), ())),
        preferred_element_type=jnp.float32)

    m = jnp.max(logits, axis=1, keepdims=True)
    ex = jnp.exp(logits - m)
    s = jnp.sum(ex, axis=1, keepdims=True)
    probs = ex / s
    probs_ref[:] = probs
    lse = m[:, 0] + jnp.log(s[:, 0])
    z_part = jnp.sum(lse * lse)
    u_part = jnp.sum(probs, axis=0)

    iota = jax.lax.broadcasted_iota(jnp.int32, logits.shape, 1)
    bits = jax.lax.bitcast_convert_type(logits, jnp.int32)
    code = jnp.where(logits >= 0.0, (_E - 1) - iota, iota)
    keys = jax.lax.bitcast_convert_type((bits & ~(_E - 1)) | code, jnp.float32)
    work = keys
    vals, inds = [], []
    for _ in range(_K):
        mk = jnp.max(work, axis=1, keepdims=True)
        vals.append(mk)
        inds.append(jax.lax.bitcast_convert_type(mk, jnp.int32) & (_E - 1))
        work = jnp.where(work == mk, -jnp.inf, work)
    v = jnp.concatenate(vals, axis=1)
    low = jnp.concatenate(inds, axis=1)
    ix = jnp.where(v >= 0.0, (_E - 1) - low, low)
    ev = jnp.exp(v - v[:, 0:1])
    wts_ref[:] = ev / jnp.sum(ev, axis=1, keepdims=True)
    idx_ref[:] = ix

    b = pl.program_id(0)

    z_blk = jnp.reshape(z_part, (1, 1))

    @pl.when(b == 0)
    def _init():
        usage_ref[:] = u_part[None, :]
        z_ref[:] = z_blk

    @pl.when(b > 0)
    def _acc():
        usage_ref[:] += u_part[None, :]
        z_ref[:] += z_blk

    @pl.when(b == pl.num_programs(0) - 1)
    def _fin():
        usage = usage_ref[:] / n_tok
        usage_ref[:] = usage
        lb_ref[:] = jnp.sum(usage * usage).reshape(1, 1) * float(_E)
        z_ref[:] = z_ref[:] / n_tok


def kernel(x, W):
    b, seq, dim = x.shape
    n_tok = b * seq
    xr = x.reshape(n_tok, dim)
    grid = (n_tok // _TBLK,)

    out = pl.pallas_call(
        functools.partial(_router_kernel, n_tok=float(n_tok)),
        grid=grid,
        in_specs=[
            pl.BlockSpec((_TBLK, dim), lambda i: (i, 0)),
            pl.BlockSpec((_E, dim), lambda i: (0, 0)),
        ],
        out_specs=[
            pl.BlockSpec((_TBLK, _K), lambda i: (i, 0)),
            pl.BlockSpec((_TBLK, _K), lambda i: (i, 0)),
            pl.BlockSpec((_TBLK, _E), lambda i: (i, 0)),
            pl.BlockSpec((1, _E), lambda i: (0, 0)),
            pl.BlockSpec((1, 1), lambda i: (0, 0)),
            pl.BlockSpec((1, 1), lambda i: (0, 0)),
        ],
        out_shape=[
            jax.ShapeDtypeStruct((n_tok, _K), jnp.float32),
            jax.ShapeDtypeStruct((n_tok, _K), jnp.int32),
            jax.ShapeDtypeStruct((n_tok, _E), jnp.float32),
            jax.ShapeDtypeStruct((1, _E), jnp.float32),
            jax.ShapeDtypeStruct((1, 1), jnp.float32),
            jax.ShapeDtypeStruct((1, 1), jnp.float32),
        ],
    )(xr, W)

    wts, idx, probs, usage, z, lb = out
    return (wts.reshape(b, seq, _K),
            idx.reshape(b, seq, _K),
            lb[0, 0],
            z[0, 0],
            usage[0],
            probs.reshape(b, seq, _E))
